# baseline (speedup 1.0000x reference)
import functools

import jax
import jax.numpy as jnp
from jax import lax
from jax.experimental import pallas as pl
from jax.experimental.pallas import tpu as pltpu
from jax.experimental.pallas import tpu_sc as plsc

C, H, W = 96, 512, 512
NC, NS = 2, 16
NW = NC * NS
RPW = H // NW
LG = W // 16
KB = 4
NBATCH = C // KB

_DN = lax.GatherDimensionNumbers(
    offset_dims=(), collapsed_slice_dims=(0,), start_index_map=(0,))


def _perm(v, idx2d):
    return lax.gather(v, idx2d, dimension_numbers=_DN, slice_sizes=(1,),
                      mode=lax.GatherScatterMode.PROMISE_IN_BOUNDS)


def _make_kernel():
    mesh = plsc.VectorSubcoreMesh(core_axis_name="c", subcore_axis_name="s")

    @functools.partial(
        pl.kernel,
        out_type=jax.ShapeDtypeStruct((C, H, W), jnp.float32),
        mesh=mesh,
        scratch_types=[
            pltpu.VMEM((2, KB, RPW, W), jnp.float32),
            pltpu.SemaphoreType.DMA,
            pltpu.SemaphoreType.DMA,
        ],
    )
    def k(act, out, buf, sem_in, sem_out):
        wid = lax.axis_index("s") * NC + lax.axis_index("c")
        r0 = wid * RPW
        iot = lax.iota(jnp.int32, 16)
        p1 = (iot ^ 1)[:, None]
        p2 = (iot ^ 2)[:, None]
        zero = jnp.zeros((16,), jnp.float32)

        def in_copy(k_):
            return pltpu.make_async_copy(
                act.at[pl.ds(KB * k_, KB), pl.ds(r0, RPW)],
                buf.at[k_ % 2], sem_in)

        def out_copy(k_):
            return pltpu.make_async_copy(
                buf.at[k_ % 2],
                out.at[pl.ds(KB * k_, KB), pl.ds(r0, RPW)], sem_out)

        def block2(bb, ci, col):
            for p in range(RPW // 2):
                a = buf[bb, ci, 2 * p, pl.ds(col, 16)]
                c = buf[bb, ci, 2 * p + 1, pl.ds(col, 16)]
                r = a + c
                s = r + _perm(r, p1)
                m = s > 0.0
                buf[bb, ci, 2 * p, pl.ds(col, 16)] = jnp.where(m, a, zero)
                buf[bb, ci, 2 * p + 1, pl.ds(col, 16)] = jnp.where(m, c, zero)

        def block4(bb, ci, col):
            for q in range(RPW // 4):
                vs = [buf[bb, ci, 4 * q + i, pl.ds(col, 16)] for i in range(4)]
                r = (vs[0] + vs[1]) + (vs[2] + vs[3])
                s2 = r + _perm(r, p1)
                s4 = s2 + _perm(s2, p2)
                m = s4 > 0.0
                for i in range(4):
                    buf[bb, ci, 4 * q + i, pl.ds(col, 16)] = jnp.where(m, vs[i], zero)

        def relu(bb, ci, col):
            for rr in range(RPW):
                v = buf[bb, ci, rr, pl.ds(col, 16)]
                buf[bb, ci, rr, pl.ds(col, 16)] = jnp.maximum(v, 0.0)

        MODES = [block2] * 8 + [block4] * 8 + [relu] * 4 + [None] * 4

        def compute(k_):
            mode = MODES[k_]
            if mode is None:
                return
            bb = k_ % 2

            def col_body(j, c2):
                for ci in range(KB):
                    mode(bb, ci, j * 16)
                return c2
            lax.fori_loop(0, LG, col_body, 0)

        in_copy(0).start()
        in_copy(0).wait()
        compute(0)
        for k_ in range(1, NBATCH):
            out_copy(k_ - 1).start()
            in_copy(k_).start()
            if k_ >= 2:
                out_copy(k_ - 2).wait()
            in_copy(k_).wait()
            compute(k_)
        out_copy(NBATCH - 1).start()
        out_copy(NBATCH - 2).wait()
        out_copy(NBATCH - 1).wait()

    return k


_k = _make_kernel()


def kernel(activation):
    act3 = activation.reshape(C, H, W)
    out = _k(act3)
    return out.reshape(1, C, H, W)


# speedup vs baseline: 1.0595x; 1.0595x over previous
import functools

import jax
import jax.numpy as jnp
from jax import lax
from jax.experimental import pallas as pl
from jax.experimental.pallas import tpu as pltpu
from jax.experimental.pallas import tpu_sc as plsc

C, H, W = 96, 512, 512
NC, NS = 2, 16
NW = NC * NS
RPW = H // NW
LG = W // 16
KB = 4
NBATCH = C // KB

_DN = lax.GatherDimensionNumbers(
    offset_dims=(), collapsed_slice_dims=(0,), start_index_map=(0,))


def _perm(v, idx2d):
    return lax.gather(v, idx2d, dimension_numbers=_DN, slice_sizes=(1,),
                      mode=lax.GatherScatterMode.PROMISE_IN_BOUNDS)


def _make_kernel():
    mesh = plsc.VectorSubcoreMesh(core_axis_name="c", subcore_axis_name="s")

    @functools.partial(
        pl.kernel,
        out_type=jax.ShapeDtypeStruct((C, H, W), jnp.float32),
        mesh=mesh,
        scratch_types=[
            pltpu.VMEM((2, KB, RPW, W), jnp.float32),
            pltpu.SemaphoreType.DMA,
            pltpu.SemaphoreType.DMA,
        ],
    )
    def k(act, out, buf, sem_in, sem_out):
        wid = lax.axis_index("s") * NC + lax.axis_index("c")
        r0 = wid * RPW
        iot = lax.iota(jnp.int32, 16)
        p1 = (iot ^ 1)[:, None]
        p2 = (iot ^ 2)[:, None]
        zero = jnp.zeros((16,), jnp.float32)

        def in_copy(k_):
            return pltpu.make_async_copy(
                act.at[pl.ds(KB * k_, KB), pl.ds(r0, RPW)],
                buf.at[k_ % 2], sem_in)

        def out_copy(k_):
            return pltpu.make_async_copy(
                buf.at[k_ % 2],
                out.at[pl.ds(KB * k_, KB), pl.ds(r0, RPW)], sem_out)

        def block2(bb, ci, col):
            for p in range(RPW // 2):
                a = buf[bb, ci, 2 * p, pl.ds(col, 16)]
                c = buf[bb, ci, 2 * p + 1, pl.ds(col, 16)]
                r = a + c
                s = r + _perm(r, p1)
                m = s > 0.0
                buf[bb, ci, 2 * p, pl.ds(col, 16)] = jnp.where(m, a, zero)
                buf[bb, ci, 2 * p + 1, pl.ds(col, 16)] = jnp.where(m, c, zero)

        def block4(bb, ci, col):
            for q in range(RPW // 4):
                vs = [buf[bb, ci, 4 * q + i, pl.ds(col, 16)] for i in range(4)]
                r = (vs[0] + vs[1]) + (vs[2] + vs[3])
                s2 = r + _perm(r, p1)
                s4 = s2 + _perm(s2, p2)
                m = s4 > 0.0
                for i in range(4):
                    buf[bb, ci, 4 * q + i, pl.ds(col, 16)] = jnp.where(m, vs[i], zero)

        def relu(bb, ci, col):
            for rr in range(RPW):
                v = buf[bb, ci, rr, pl.ds(col, 16)]
                buf[bb, ci, rr, pl.ds(col, 16)] = jnp.maximum(v, 0.0)

        MODES = [block2] * 8 + [block4] * 8 + [relu] * 4 + [None] * 4

        def compute(k_):
            mode = MODES[k_]
            if mode is None:
                return
            bb = k_ % 2

            def col_body(j, c2):
                for ci in range(KB):
                    mode(bb, ci, j * 16)
                return c2
            lax.fori_loop(0, LG, col_body, 0)

        in_copy(0).start()
        in_copy(0).wait()
        compute(0)
        in_copy(1).start()
        for k_ in range(1, NBATCH):
            in_copy(k_).wait()
            if k_ >= 2:
                out_copy(k_ - 2).wait()
            out_copy(k_ - 1).start()
            compute(k_)
            if k_ + 1 < NBATCH:
                in_copy(k_ + 1).start()
        out_copy(NBATCH - 1).start()
        out_copy(NBATCH - 2).wait()
        out_copy(NBATCH - 1).wait()

    return k


_k = _make_kernel()


def kernel(activation):
    act3 = activation.reshape(C, H, W)
    out = _k(act3)
    return out.reshape(1, C, H, W)


# speedup vs baseline: 1.1284x; 1.0650x over previous
import functools

import jax
import jax.numpy as jnp
from jax import lax
from jax.experimental import pallas as pl
from jax.experimental.pallas import tpu as pltpu
from jax.experimental.pallas import tpu_sc as plsc

C, H, W = 96, 512, 512
NC, NS = 2, 16
NW = NC * NS
RPW = H // NW
LG = W // 16
KBMAX = 7


def _mk_batches():
    bl = []
    for base, n, tag in ((0, 32, "b2"), (32, 32, "b4"),
                         (64, 16, "relu"), (80, 16, None)):
        c = 0
        while c < n:
            kb = min(KBMAX, n - c)
            bl.append((base + c, kb, tag))
            c += kb
    return bl


BATCHES = _mk_batches()
NBATCH = len(BATCHES)

_DN = lax.GatherDimensionNumbers(
    offset_dims=(), collapsed_slice_dims=(0,), start_index_map=(0,))


def _perm(v, idx2d):
    return lax.gather(v, idx2d, dimension_numbers=_DN, slice_sizes=(1,),
                      mode=lax.GatherScatterMode.PROMISE_IN_BOUNDS)


def _make_kernel():
    mesh = plsc.VectorSubcoreMesh(core_axis_name="c", subcore_axis_name="s")

    @functools.partial(
        pl.kernel,
        out_type=jax.ShapeDtypeStruct((C, H, W), jnp.float32),
        mesh=mesh,
        scratch_types=[
            pltpu.VMEM((2, KBMAX, RPW, W), jnp.float32),
            pltpu.SemaphoreType.DMA,
            pltpu.SemaphoreType.DMA,
        ],
    )
    def k(act, out, buf, sem_in, sem_out):
        wid = lax.axis_index("s") * NC + lax.axis_index("c")
        r0 = wid * RPW
        iot = lax.iota(jnp.int32, 16)
        p1 = (iot ^ 1)[:, None]
        p2 = (iot ^ 2)[:, None]
        zero = jnp.zeros((16,), jnp.float32)

        def in_copy(k_):
            c0, kb, _ = BATCHES[k_]
            return pltpu.make_async_copy(
                act.at[pl.ds(c0, kb), pl.ds(r0, RPW)],
                buf.at[k_ % 2, pl.ds(0, kb)], sem_in)

        def out_copy(k_):
            c0, kb, _ = BATCHES[k_]
            return pltpu.make_async_copy(
                buf.at[k_ % 2, pl.ds(0, kb)],
                out.at[pl.ds(c0, kb), pl.ds(r0, RPW)], sem_out)

        def block2(bb, ci, col):
            for p in range(RPW // 2):
                a = buf[bb, ci, 2 * p, pl.ds(col, 16)]
                c = buf[bb, ci, 2 * p + 1, pl.ds(col, 16)]
                r = a + c
                s = r + _perm(r, p1)
                m = s > 0.0
                buf[bb, ci, 2 * p, pl.ds(col, 16)] = jnp.where(m, a, zero)
                buf[bb, ci, 2 * p + 1, pl.ds(col, 16)] = jnp.where(m, c, zero)

        def block4(bb, ci, col):
            for q in range(RPW // 4):
                vs = [buf[bb, ci, 4 * q + i, pl.ds(col, 16)] for i in range(4)]
                r = (vs[0] + vs[1]) + (vs[2] + vs[3])
                s2 = r + _perm(r, p1)
                s4 = s2 + _perm(s2, p2)
                m = s4 > 0.0
                for i in range(4):
                    buf[bb, ci, 4 * q + i, pl.ds(col, 16)] = jnp.where(m, vs[i], zero)

        def relu(bb, ci, col):
            for rr in range(RPW):
                v = buf[bb, ci, rr, pl.ds(col, 16)]
                buf[bb, ci, rr, pl.ds(col, 16)] = jnp.maximum(v, 0.0)

        MODEFN = {"b2": block2, "b4": block4, "relu": relu}

        def compute(k_):
            _, kb, tag = BATCHES[k_]
            if tag is None:
                return
            mode = MODEFN[tag]
            bb = k_ % 2

            def col_body(j, c2):
                for ci in range(kb):
                    mode(bb, ci, j * 16)
                return c2
            lax.fori_loop(0, LG, col_body, 0)

        in_copy(0).start()
        in_copy(0).wait()
        compute(0)
        in_copy(1).start()
        for k_ in range(1, NBATCH):
            in_copy(k_).wait()
            if k_ >= 2:
                out_copy(k_ - 2).wait()
            out_copy(k_ - 1).start()
            compute(k_)
            if k_ + 1 < NBATCH:
                in_copy(k_ + 1).start()
        out_copy(NBATCH - 1).start()
        out_copy(NBATCH - 2).wait()
        out_copy(NBATCH - 1).wait()

    return k


_k = _make_kernel()


def kernel(activation):
    act3 = activation.reshape(C, H, W)
    out = _k(act3)
    return out.reshape(1, C, H, W)
